# baseline (speedup 1.0000x reference)
import functools

import jax
import jax.numpy as jnp
from jax import lax
from jax.experimental import pallas as pl
from jax.experimental.pallas import tpu as pltpu
from jax.experimental.pallas import tpu_sc as plsc

_INFO = plsc.get_sparse_core_info()
_NC = _INFO.num_cores
_NS = _INFO.num_subcores
_NW = _NC * _NS


_SPLIT = 2
_NBUF = 12
_AHEAD = 6


@functools.partial(jax.jit, static_argnames=("b", "s", "d_model"))
def _gather(weights, xt, b, s, d_model):
    per_w = b // _NW
    ch = per_w // _SPLIT
    n_chunk = s * _SPLIT
    nbuf, ahead = _NBUF, _AHEAD
    mesh = plsc.VectorSubcoreMesh(core_axis_name="c", subcore_axis_name="s")

    @functools.partial(
        pl.kernel,
        out_type=jax.ShapeDtypeStruct((s, b, d_model), jnp.float32),
        mesh=mesh,
        compiler_params=pltpu.CompilerParams(use_tc_tiling_on_sc=False),
        scratch_types=[
            pltpu.VMEM((s, per_w), jnp.int32),
            pltpu.VMEM((nbuf, ch, d_model), jnp.float32),
            [pltpu.SemaphoreType.DMA] * nbuf,
            [pltpu.SemaphoreType.DMA] * nbuf,
        ],
    )
    def body(table_hbm, idx_hbm, out_hbm, idx_v, rows_v, gsem, osem):
        wid = lax.axis_index("s") * _NC + lax.axis_index("c")
        base = wid * per_w
        pltpu.sync_copy(idx_hbm.at[pl.ds(0, s), pl.ds(base, per_w)], idx_v)

        def start_gather(c, bf):
            j, h = c // _SPLIT, c % _SPLIT
            pltpu.async_copy(
                table_hbm.at[idx_v.at[j, pl.ds(h * ch, ch)]],
                rows_v.at[bf], gsem[bf])

        def wait_gather(c, bf):
            j, h = c // _SPLIT, c % _SPLIT
            pltpu.make_async_copy(
                table_hbm.at[idx_v.at[j, pl.ds(h * ch, ch)]],
                rows_v.at[bf], gsem[bf]).wait()

        def start_out(c, bf):
            j, h = c // _SPLIT, c % _SPLIT
            pltpu.async_copy(
                rows_v.at[bf],
                out_hbm.at[j, pl.ds(base + h * ch, ch)], osem[bf])

        def wait_out(bf):
            pltpu.make_async_copy(
                rows_v.at[bf], out_hbm.at[0, pl.ds(base, ch)], osem[bf]).wait()

        for g in range(ahead):
            start_gather(g, g)
        for c in range(ahead):
            wait_gather(c, c)
            start_out(c, c)
            start_gather(c + ahead, c + ahead)

        def step(t, carry):
            c0 = nbuf * t + ahead
            for k in range(nbuf):
                c = c0 + k
                bf = (ahead + k) % nbuf
                bn = (ahead + k + ahead) % nbuf
                wait_gather(c, bf)
                start_out(c, bf)
                wait_out(bn)
                start_gather(c + ahead, bn)
            return carry

        n_loop = (n_chunk - 2 * ahead) // nbuf
        lax.fori_loop(0, n_loop, step, 0)

        for c in range(nbuf * n_loop + ahead, n_chunk - ahead):
            bf = c % nbuf
            wait_gather(c, bf)
            start_out(c, bf)
            wait_out((c + ahead) % nbuf)
            start_gather(c + ahead, (c + ahead) % nbuf)
        for c in range(n_chunk - ahead, n_chunk):
            bf = c % nbuf
            wait_gather(c, bf)
            start_out(c, bf)
        for bf in range(nbuf):
            wait_out(bf)

    return body(weights, xt)


def kernel(x, weights):
    b, s = x.shape
    d_model = weights.shape[1]
    out = _gather(weights, x.T, b, s, d_model)
    return lax.stop_gradient(jnp.transpose(out, (1, 0, 2)))
